# Initial kernel scaffold; baseline (speedup 1.0000x reference)
import jax
import jax.numpy as jnp
from jax.experimental import pallas as pl


def kernel(logits, inpData):
    raise NotImplementedError("write your pallas kernel here")


# speedup vs baseline: 18.5585x; 18.5585x over previous
import jax
import jax.numpy as jnp
from jax import lax
from jax.experimental import pallas as pl

_MUX_IN = 32768
_MUX_OUT = 16
_R = 256
_C = 128


def _topk_onehot_body(logits_ref, gn_ref, out_ref):
    x = logits_ref[...] + gn_ref[0]
    row_iota = lax.broadcasted_iota(jnp.int32, (_R, _C), 0)
    col_iota = lax.broadcasted_iota(jnp.int32, (_R, _C), 1)
    iota = row_iota * _C + col_iota

    def step(k, x):
        m = jnp.max(x)
        idx = jnp.min(jnp.where(x == m, iota, jnp.int32(2**30)))
        hit = iota == idx
        out_ref[0, pl.ds(k, 1), :, :] = hit.astype(jnp.float32)[None, :, :]
        return jnp.where(hit, -jnp.inf, x)

    lax.fori_loop(0, _MUX_OUT, step, x)


def kernel(logits, inpData):
    BS = inpData.shape[0]
    u = jax.random.uniform(
        jax.random.key(42), (BS, 1, _MUX_IN), minval=0.0, maxval=1.0,
        dtype=jnp.float32)
    gn = (-jnp.log(-jnp.log(u + 1e-20) + 1e-20)).reshape(BS, _R, _C)
    logits2d = logits.reshape(_R, _C)

    out = pl.pallas_call(
        _topk_onehot_body,
        grid=(BS,),
        in_specs=[
            pl.BlockSpec((_R, _C), lambda b: (0, 0)),
            pl.BlockSpec((1, _R, _C), lambda b: (b, 0, 0)),
        ],
        out_specs=pl.BlockSpec((1, _MUX_OUT, _R, _C), lambda b: (b, 0, 0, 0)),
        out_shape=jax.ShapeDtypeStruct((BS, _MUX_OUT, _R, _C), jnp.float32),
    )(logits2d, gn)
    return out.reshape(BS, _MUX_OUT, _MUX_IN)


# speedup vs baseline: 45.3296x; 2.4425x over previous
import jax
import jax.numpy as jnp
from jax import lax
from jax.experimental import pallas as pl
from jax.experimental.pallas import tpu as pltpu

_MUX_IN = 32768
_MUX_OUT = 16


def _topk_body(logits_ref, u_ref, idx_ref):
    u = u_ref[...]
    gn = -jnp.log(-jnp.log(u + 1e-20) + 1e-20)
    x = logits_ref[...] + gn
    bs = x.shape[0]
    iota = lax.broadcasted_iota(jnp.int32, (bs, _MUX_IN), 1)
    cols = []
    for _ in range(_MUX_OUT):
        m = jnp.max(x, axis=1, keepdims=True)
        idx = jnp.min(jnp.where(x == m, iota, jnp.int32(2**30)),
                      axis=1, keepdims=True)
        cols.append(idx)
        x = jnp.where(iota == idx, -jnp.inf, x)
    idx_ref[...] = jnp.concatenate(cols, axis=1)


def _onehot_body(idx_sref, out_ref):
    i = pl.program_id(0)
    iota = lax.broadcasted_iota(jnp.int32, (1, _MUX_IN), 1)
    for j in range(_MUX_OUT):
        idx = idx_sref[i * _MUX_OUT + j]
        out_ref[pl.ds(j, 1), :] = (iota == idx).astype(jnp.float32)


def kernel(logits, inpData):
    BS = inpData.shape[0]
    u = jax.random.uniform(
        jax.random.key(42), (BS, _MUX_IN), minval=0.0, maxval=1.0,
        dtype=jnp.float32)

    topk_idx = pl.pallas_call(
        _topk_body,
        grid=(1,),
        in_specs=[
            pl.BlockSpec((1, _MUX_IN), lambda i: (0, 0)),
            pl.BlockSpec((BS, _MUX_IN), lambda i: (0, 0)),
        ],
        out_specs=pl.BlockSpec((BS, _MUX_OUT), lambda i: (0, 0)),
        out_shape=jax.ShapeDtypeStruct((BS, _MUX_OUT), jnp.int32),
    )(logits, u)

    out = pl.pallas_call(
        _onehot_body,
        grid_spec=pltpu.PrefetchScalarGridSpec(
            num_scalar_prefetch=1,
            grid=(BS,),
            in_specs=[],
            out_specs=pl.BlockSpec((_MUX_OUT, _MUX_IN),
                                   lambda i, idx_ref: (i, 0)),
        ),
        out_shape=jax.ShapeDtypeStruct((BS * _MUX_OUT, _MUX_IN), jnp.float32),
    )(topk_idx.reshape(-1))
    return out.reshape(BS, _MUX_OUT, _MUX_IN)


# speedup vs baseline: 46.6660x; 1.0295x over previous
import jax
import jax.numpy as jnp
from jax import lax
from jax.experimental import pallas as pl
from jax.experimental.pallas import tpu as pltpu

_MUX_IN = 32768
_MUX_OUT = 16


def _topk_body(logits_ref, gn_ref, idx_ref):
    x = logits_ref[...] + gn_ref[...]
    iota = lax.broadcasted_iota(jnp.int32, x.shape, 1)
    cols = []
    for _ in range(_MUX_OUT):
        idx = jnp.argmax(x, axis=1).astype(jnp.int32)[:, None]
        cols.append(idx)
        x = jnp.where(iota == idx, -jnp.inf, x)
    idx_ref[...] = jnp.concatenate(cols, axis=1)


def _onehot_body(idx_sref, out_ref):
    i = pl.program_id(0)
    iota = lax.broadcasted_iota(jnp.int32, (1, _MUX_IN), 1)
    for j in range(_MUX_OUT):
        idx = idx_sref[i * _MUX_OUT + j]
        out_ref[pl.ds(j, 1), :] = (iota == idx).astype(jnp.float32)


def kernel(logits, inpData):
    BS = inpData.shape[0]
    u = jax.random.uniform(
        jax.random.key(42), (BS, _MUX_IN), minval=0.0, maxval=1.0,
        dtype=jnp.float32)
    gn = -jnp.log(-jnp.log(u + 1e-20) + 1e-20)

    topk_idx = pl.pallas_call(
        _topk_body,
        grid=(1,),
        in_specs=[
            pl.BlockSpec((1, _MUX_IN), lambda i: (0, 0)),
            pl.BlockSpec((BS, _MUX_IN), lambda i: (0, 0)),
        ],
        out_specs=pl.BlockSpec((BS, _MUX_OUT), lambda i: (0, 0)),
        out_shape=jax.ShapeDtypeStruct((BS, _MUX_OUT), jnp.int32),
    )(logits, gn)

    out = pl.pallas_call(
        _onehot_body,
        grid_spec=pltpu.PrefetchScalarGridSpec(
            num_scalar_prefetch=1,
            grid=(BS,),
            in_specs=[],
            out_specs=pl.BlockSpec((_MUX_OUT, _MUX_IN),
                                   lambda i, idx_ref: (i, 0)),
        ),
        out_shape=jax.ShapeDtypeStruct((BS * _MUX_OUT, _MUX_IN), jnp.float32),
    )(topk_idx.reshape(-1))
    return out.reshape(BS, _MUX_OUT, _MUX_IN)
